# baseline (speedup 1.0000x reference)
import functools

import jax
import jax.numpy as jnp
from jax import lax
from jax.experimental import pallas as pl
from jax.experimental.pallas import tpu as pltpu
from jax.experimental.pallas import tpu_sc as plsc

L = 16
CH = 128


def _tab_body(x_ref, wb_ref, ws_ref, ab_ref, as_ref, tabb_ref, tabs_ref, hv_ref):
    xb = x_ref[...]
    hb = jnp.dot(xb, wb_ref[...], preferred_element_type=jnp.float32)
    hs = jnp.dot(xb, ws_ref[...], preferred_element_type=jnp.float32)
    blk = xb.shape[0]
    pb = jnp.dot(hb, ab_ref[...], preferred_element_type=jnp.float32)
    ps = jnp.dot(hs, as_ref[...], preferred_element_type=jnp.float32)
    lane = lax.broadcasted_iota(jnp.int32, (blk, L), 1)
    ones_col = (lane == 0).astype(jnp.float32)
    hs_lane = (lane == 1).astype(jnp.float32)
    tabb_ref[...] = jnp.concatenate(
        [hb, ones_col + pb[:, 0:1] * hs_lane], axis=1)
    tabs_ref[...] = jnp.concatenate(
        [hs, ones_col + ps[:, 0:1] * hs_lane], axis=1)
    hv_ref[...] = (pb[:, 0:1] * (lane == 0).astype(jnp.float32)
                   + pb[:, 1:2] * hs_lane
                   + ps[:, 0:1] * (lane == 2).astype(jnp.float32)
                   + ps[:, 1:2] * (lane == 3).astype(jnp.float32))


def _build_tables(x2d, wb, ws, ab, as_):
    R, F = x2d.shape
    BLK = 2000
    grid = (R // BLK,)
    W = F + L
    return pl.pallas_call(
        _tab_body,
        grid=grid,
        in_specs=[
            pl.BlockSpec((BLK, F), lambda i: (i, 0)),
            pl.BlockSpec((F, F), lambda i: (0, 0)),
            pl.BlockSpec((F, F), lambda i: (0, 0)),
            pl.BlockSpec((F, 2), lambda i: (0, 0)),
            pl.BlockSpec((F, 2), lambda i: (0, 0)),
        ],
        out_specs=[
            pl.BlockSpec((BLK, W), lambda i: (i, 0)),
            pl.BlockSpec((BLK, W), lambda i: (i, 0)),
            pl.BlockSpec((BLK, L), lambda i: (i, 0)),
        ],
        out_shape=[
            jax.ShapeDtypeStruct((R, W), jnp.float32),
            jax.ShapeDtypeStruct((R, W), jnp.float32),
            jax.ShapeDtypeStruct((R, L), jnp.float32),
        ],
    )(x2d, wb, ws, ab, as_)


def _et_body(ea_ref, v5_ref, out_ref, *, ep, ev):
    ea = ea_ref[...]
    et = jnp.sum(ea * v5_ref[...], axis=1)
    et2 = et.reshape(8, CH)
    pid = pl.program_id(0)
    g = (pid * 1024
         + lax.broadcasted_iota(jnp.int32, (8, CH), 0) * CH
         + lax.broadcasted_iota(jnp.int32, (8, CH), 1))
    valid = (g % ep) < ev
    out_ref[...] = jnp.where(valid, et2, -1e30)


def _edge_terms(ea5, v5, ep, ev):
    Ne = ea5.shape[0]
    grid = (Ne // 1024,)
    return pl.pallas_call(
        functools.partial(_et_body, ep=ep, ev=ev),
        grid=grid,
        in_specs=[
            pl.BlockSpec((1024, 5), lambda i: (i, 0)),
            pl.BlockSpec((1, 5), lambda i: (0, 0)),
        ],
        out_specs=pl.BlockSpec((8, CH), lambda i: (i, 0)),
        out_shape=jax.ShapeDtypeStruct((Ne // CH, CH), jnp.float32),
    )(ea5, v5)


def _make_sc_kernel(B, N, NPS, C, KB, KS):
    W = C + L
    NT = 16
    NPT = 640
    N2 = NT * NPT
    NCHK = 64
    mesh = plsc.VectorSubcoreMesh(core_axis_name="c", subcore_axis_name="s")

    @functools.partial(
        pl.kernel,
        out_type=jax.ShapeDtypeStruct((B * N2, C), jnp.float32),
        mesh=mesh,
        compiler_params=pltpu.CompilerParams(needs_layout_passes=False,
                                             use_tc_tiling_on_sc=False),
        scratch_types=[
            pltpu.VMEM_SHARED((N2, W), jnp.float32),
            pltpu.VMEM((CH,), jnp.int32),
            pltpu.VMEM((CH,), jnp.int32),
            pltpu.VMEM((CH,), jnp.float32),
            pltpu.VMEM((2, CH // 2), jnp.int32),
            pltpu.VMEM((2, CH // 2), jnp.int32),
            pltpu.VMEM((CH,), jnp.int32),
            pltpu.VMEM((CH,), jnp.int32),
            pltpu.VMEM((CH,), jnp.float32),
            pltpu.VMEM((CH, L), jnp.float32),
            pltpu.VMEM((CH, L), jnp.float32),
            pltpu.VMEM((CH // 2, W), jnp.float32),
            pltpu.VMEM((CH // 2, W), jnp.float32),
            pltpu.VMEM((NCHK, C), jnp.float32),
            pltpu.VMEM((2 * C + 2 * L,), jnp.float32),
            pltpu.SemaphoreType.DMA,
            pltpu.SemaphoreType.DMA,
            pltpu.SemaphoreType.DMA,
            pltpu.SemaphoreType.DMA,
        ],
    )
    def sc_kernel(tabb, tabs, hv_h, etb, ets,
                  bsrc, bdst, ssrc, sdst, par_h, out_h,
                  acc, sbuf, dbuf, ebuf, gbuf2, dbuf2, sgbuf, dgbuf, exbuf,
                  srow, drow, rowsA, rowsB, obuf, pbuf, sem, sem2, sem3, sem4):
        b = lax.axis_index("c")
        t = lax.axis_index("s")
        bN = b * N
        bN2 = b * N2

        pltpu.sync_copy(par_h, pbuf)

        def zero_rows():
            def zb(i, _):
                for c9 in range(W // L):
                    rowsA[i, pl.ds(L * c9, L)] = jnp.zeros((L,), jnp.float32)
                return 0
            lax.fori_loop(0, NCHK, zb, 0)

        def zero_acc():
            def zc(j, _):
                pltpu.sync_copy(rowsA,
                                acc.at[pl.ds(t * NPT + j * NCHK, NCHK)])
                return 0
            lax.fori_loop(0, NPT // NCHK, zc, 0)

        def edge_phase(tab_ref, et_ref, src2d, dst2d, hs_col,
                       nchunks, sec_off, src_row0, et_row0):
            H = CH // 2

            def issue_src_dst(k):
                pltpu.async_copy(src2d.at[src_row0 + k], sbuf, sem4)
                pltpu.async_copy(dst2d.at[src_row0 + k], dbuf, sem4)

            def issue_et(k):
                pltpu.async_copy(et_ref.at[et_row0 + k], ebuf, sem4)

            def wait_linear():
                pltpu.make_async_copy(src2d.at[src_row0], sbuf, sem4).wait()
                pltpu.make_async_copy(dst2d.at[src_row0], dbuf, sem4).wait()
                pltpu.make_async_copy(et_ref.at[et_row0], ebuf, sem4).wait()

            def scale_half(rowsX, off):
                for j in range(H // L):
                    idr = lax.iota(jnp.int32, L) + L * j
                    exv16 = exbuf[pl.ds(off + L * j, L)]
                    plsc.store_scatter(
                        rowsX, [idr, jnp.full((L,), C, jnp.int32)], exv16)

                @plsc.parallel_loop(0, H, unroll=8)
                def _(e):
                    exv = plsc.load_gather(
                        exbuf, [jnp.full((L,), e + off, jnp.int32)])
                    for c9 in range(C // L):
                        sl2 = pl.ds(L * c9, L)
                        rowsX[e, sl2] = rowsX[e, sl2] * exv

            def wait_scatters():
                pltpu.make_async_copy(rowsA, acc.at[dbuf2.at[0]], sem).wait()
                pltpu.make_async_copy(rowsB, acc.at[dbuf2.at[1]], sem2).wait()

            issue_src_dst(0)
            issue_et(0)

            def chunk(k, _):
                @pl.when(k > 0)
                def _():
                    wait_scatters()
                wait_linear()
                for j in range(CH // L):
                    sl = pl.ds(L * j, L)
                    si = sbuf[sl] + sec_off
                    di = dbuf[sl] + sec_off
                    gbuf2[j // 4, pl.ds(L * (j % 4), L)] = si + bN
                    dbuf2[j // 4, pl.ds(L * (j % 4), L)] = di
                    sgbuf[sl] = si + bN
                    dgbuf[sl] = di + bN
                gA = pltpu.async_copy(tab_ref.at[gbuf2.at[0]], rowsA, sem)
                gB = pltpu.async_copy(tab_ref.at[gbuf2.at[1]], rowsB, sem2)
                gs = pltpu.async_copy(hv_h.at[sgbuf], srow, sem3)
                gd = pltpu.async_copy(hv_h.at[dgbuf], drow, sem3)

                @pl.when(k + 1 < nchunks)
                def _():
                    issue_src_dst(k + 1)
                gs.wait()
                gd.wait()
                for j in range(CH // L):
                    sl = pl.ds(L * j, L)
                    idr = lax.iota(jnp.int32, L) + L * j
                    hsv = plsc.load_gather(
                        srow, [idr, jnp.full((L,), hs_col, jnp.int32)])
                    hdv = plsc.load_gather(
                        drow, [idr, jnp.full((L,), hs_col + 1, jnp.int32)])
                    al = hsv + hdv + ebuf[sl]
                    al = jnp.maximum(al, 0.2 * al)
                    exbuf[sl] = jnp.exp(al)

                @pl.when(k + 1 < nchunks)
                def _():
                    issue_et(k + 1)
                gA.wait()
                scale_half(rowsA, 0)
                pltpu.async_copy(rowsA, acc.at[dbuf2.at[0]], sem, add=True)
                gB.wait()
                scale_half(rowsB, H)
                pltpu.async_copy(rowsB, acc.at[dbuf2.at[1]], sem2, add=True)
                return 0
            lax.fori_loop(0, nchunks, chunk, 0)
            wait_scatters()

        def finalize(is_bend):
            def fch(jj, _):
                n0 = t * NPT + jj * NCHK
                pltpu.sync_copy(acc.at[pl.ds(n0, NCHK)], rowsA)
                if not is_bend:
                    pltpu.sync_copy(out_h.at[pl.ds(bN2 + n0, NCHK)], obuf)

                @plsc.parallel_loop(0, NCHK, unroll=2)
                def fn(n):
                    denv = plsc.load_gather(
                        rowsA, [jnp.full((L,), n, jnp.int32),
                                jnp.full((L,), C, jnp.int32)])
                    rcp = 1.0 / (denv + 1e-16)
                    for c9 in range(C // L):
                        sl = pl.ds(L * c9, L)
                        bias = pbuf[pl.ds((0 if is_bend else C) + L * c9, L)]
                        v = rowsA[n, sl] * rcp + bias
                        if is_bend:
                            r = jnp.maximum(v, 0.01 * v) * pbuf[pl.ds(2 * C, L)]
                            obuf[n, sl] = r
                        else:
                            obuf[n, sl] = obuf[n, sl] + v * pbuf[pl.ds(2 * C + L, L)]
                pltpu.sync_copy(obuf, out_h.at[pl.ds(bN2 + n0, NCHK)])
                return 0
            lax.fori_loop(0, NPT // NCHK, fch, 0)

        zero_rows()
        zero_acc()
        plsc.subcore_barrier()
        edge_phase(tabb, etb, bsrc, bdst, 0,
                   KB, jnp.int32(0), t * KB, b * (KB * NT) + t * KB)
        plsc.subcore_barrier()
        finalize(True)
        zero_rows()
        zero_acc()
        plsc.subcore_barrier()
        part = t % 2
        sec = t // 2
        edge_phase(tabs, ets, ssrc, sdst, 2,
                   KS, sec * NPS, part * KS, b * (KS * 2) + part * KS)
        plsc.subcore_barrier()
        finalize(False)

    return sc_kernel


def kernel(x, section_edge_index, bend_edge_index, section_edge_attr,
           bend_edge_attr, enc_W, enc_b, gatb_W, gatb_as, gatb_ad, gatb_ae,
           gatb_We, gatb_bias, gats_W, gats_as, gats_ad, gats_ae, gats_We,
           gats_bias, mix_w):
    B, S, NPS, F = x.shape
    N = S * NPS
    C = gatb_W.shape[1]
    ES = section_edge_index.shape[1]
    EB = bend_edge_index.shape[1]
    NT = 16
    KB = -(-(-(-EB // (NT * CH))) // 8) * 8
    EBp = KB * NT * CH
    KS = -(-(-(-ES // (2 * CH))) // 8) * 8
    ESp_half = KS * 2 * CH
    while (B * EBp) % 1024:
        KB += 1
        EBp = KB * NT * CH
    while (B * ESp_half) % 1024:
        KS += 1
        ESp_half = KS * 2 * CH
    ESp = ESp_half

    f32 = jnp.float32
    veb = enc_W @ (gatb_We @ gatb_ae)
    ceb = enc_b @ (gatb_We @ gatb_ae)
    ves = enc_W @ (gats_We @ gats_ae)
    ces = enc_b @ (gats_We @ gats_ae)
    v5b = jnp.concatenate([veb, ceb[None]])[None, :]
    v5s = jnp.concatenate([ves, ces[None]])[None, :]
    ab = jnp.concatenate([gatb_as[:, None], gatb_ad[:, None]], axis=1)
    as_ = jnp.concatenate([gats_as[:, None], gats_ad[:, None]], axis=1)
    w = jax.nn.softmax(mix_w)
    params = jnp.concatenate([gatb_bias, gats_bias,
                              jnp.full((L,), w[0], f32),
                              jnp.full((L,), w[1], f32)])

    x2d = x.reshape(B * N, F)
    tabb, tabs, hv = _build_tables(x2d, gatb_W, gats_W, ab, as_)

    def pad_ea(ea, Ep):
        E = ea.shape[1]
        eap = jnp.pad(ea, ((0, 0), (0, Ep - E), (0, 0)))
        flat = eap.reshape(B * Ep, 4)
        return jnp.concatenate([flat, jnp.ones((B * Ep, 1), f32)], axis=1)
    etb = _edge_terms(pad_ea(bend_edge_attr, EBp), v5b, EBp, EB)
    ets = _edge_terms(pad_ea(section_edge_attr, ESp), v5s, ESp, ES)

    def pad_idx(idx, Ep):
        E = idx.shape[0]
        return jnp.pad(idx, (0, Ep - E)).astype(jnp.int32).reshape(Ep // CH, CH)
    bsrc = pad_idx(bend_edge_index[0], EBp)
    bdst = pad_idx(bend_edge_index[1], EBp)
    ssrc = pad_idx(section_edge_index[0], ESp)
    sdst = pad_idx(section_edge_index[1], ESp)

    sc = _make_sc_kernel(B, N, NPS, C, KB, KS)
    out = sc(tabb, tabs, hv, etb, ets,
             bsrc, bdst, ssrc, sdst, params)
    N2 = out.shape[0] // B
    return out.reshape(B, N2, C)[:, :N]


# speedup vs baseline: 1.0183x; 1.0183x over previous
import functools

import jax
import jax.numpy as jnp
from jax import lax
from jax.experimental import pallas as pl
from jax.experimental.pallas import tpu as pltpu
from jax.experimental.pallas import tpu_sc as plsc

L = 16
CH = 128


def _tab_body(x_ref, wb_ref, ws_ref, ab_ref, as_ref, tabb_ref, tabs_ref, hv_ref):
    xb = x_ref[...]
    hb = jnp.dot(xb, wb_ref[...], preferred_element_type=jnp.float32)
    hs = jnp.dot(xb, ws_ref[...], preferred_element_type=jnp.float32)
    blk = xb.shape[0]
    pb = jnp.dot(hb, ab_ref[...], preferred_element_type=jnp.float32)
    ps = jnp.dot(hs, as_ref[...], preferred_element_type=jnp.float32)
    lane = lax.broadcasted_iota(jnp.int32, (blk, L), 1)
    ones_col = (lane == 0).astype(jnp.float32)
    hs_lane = (lane == 1).astype(jnp.float32)
    tabb_ref[...] = jnp.concatenate(
        [hb, ones_col + pb[:, 0:1] * hs_lane], axis=1)
    tabs_ref[...] = jnp.concatenate(
        [hs, ones_col + ps[:, 0:1] * hs_lane], axis=1)
    hv_ref[...] = (pb[:, 0:1] * (lane == 0).astype(jnp.float32)
                   + pb[:, 1:2] * hs_lane
                   + ps[:, 0:1] * (lane == 2).astype(jnp.float32)
                   + ps[:, 1:2] * (lane == 3).astype(jnp.float32))


def _build_tables(x2d, wb, ws, ab, as_):
    R, F = x2d.shape
    BLK = 2000
    grid = (R // BLK,)
    W = F + L
    return pl.pallas_call(
        _tab_body,
        grid=grid,
        in_specs=[
            pl.BlockSpec((BLK, F), lambda i: (i, 0)),
            pl.BlockSpec((F, F), lambda i: (0, 0)),
            pl.BlockSpec((F, F), lambda i: (0, 0)),
            pl.BlockSpec((F, 2), lambda i: (0, 0)),
            pl.BlockSpec((F, 2), lambda i: (0, 0)),
        ],
        out_specs=[
            pl.BlockSpec((BLK, W), lambda i: (i, 0)),
            pl.BlockSpec((BLK, W), lambda i: (i, 0)),
            pl.BlockSpec((BLK, L), lambda i: (i, 0)),
        ],
        out_shape=[
            jax.ShapeDtypeStruct((R, W), jnp.float32),
            jax.ShapeDtypeStruct((R, W), jnp.float32),
            jax.ShapeDtypeStruct((R, L), jnp.float32),
        ],
    )(x2d, wb, ws, ab, as_)


def _et_body(ea_ref, v5_ref, out_ref, *, ep, ev):
    ea = ea_ref[...]
    et = jnp.sum(ea * v5_ref[...], axis=1)
    et2 = et.reshape(8, CH)
    pid = pl.program_id(0)
    g = (pid * 1024
         + lax.broadcasted_iota(jnp.int32, (8, CH), 0) * CH
         + lax.broadcasted_iota(jnp.int32, (8, CH), 1))
    valid = (g % ep) < ev
    out_ref[...] = jnp.where(valid, et2, -1e30)


def _edge_terms(ea5, v5, ep, ev):
    Ne = ea5.shape[0]
    grid = (Ne // 1024,)
    return pl.pallas_call(
        functools.partial(_et_body, ep=ep, ev=ev),
        grid=grid,
        in_specs=[
            pl.BlockSpec((1024, 5), lambda i: (i, 0)),
            pl.BlockSpec((1, 5), lambda i: (0, 0)),
        ],
        out_specs=pl.BlockSpec((8, CH), lambda i: (i, 0)),
        out_shape=jax.ShapeDtypeStruct((Ne // CH, CH), jnp.float32),
    )(ea5, v5)


def _make_sc_kernel(B, N, NPS, C, KB, KS):
    W = C + L
    NT = 16
    NPT = 640
    N2 = NT * NPT
    NCHK = 64
    mesh = plsc.VectorSubcoreMesh(core_axis_name="c", subcore_axis_name="s")

    @functools.partial(
        pl.kernel,
        out_type=jax.ShapeDtypeStruct((B * N2, C), jnp.float32),
        mesh=mesh,
        compiler_params=pltpu.CompilerParams(needs_layout_passes=False,
                                             use_tc_tiling_on_sc=False),
        scratch_types=[
            pltpu.VMEM_SHARED((N2, W), jnp.float32),
            pltpu.VMEM((CH,), jnp.int32),
            pltpu.VMEM((CH,), jnp.int32),
            pltpu.VMEM((CH,), jnp.float32),
            pltpu.VMEM((2, CH // 2), jnp.int32),
            pltpu.VMEM((2, CH // 2), jnp.int32),
            pltpu.VMEM((CH,), jnp.int32),
            pltpu.VMEM((CH,), jnp.int32),
            pltpu.VMEM((CH,), jnp.float32),
            pltpu.VMEM((CH, L), jnp.float32),
            pltpu.VMEM((CH, L), jnp.float32),
            pltpu.VMEM((CH // 2, W), jnp.float32),
            pltpu.VMEM((CH // 2, W), jnp.float32),
            pltpu.VMEM((NCHK, C), jnp.float32),
            pltpu.VMEM((2 * C + 2 * L,), jnp.float32),
            pltpu.SemaphoreType.DMA,
            pltpu.SemaphoreType.DMA,
            pltpu.SemaphoreType.DMA,
            pltpu.SemaphoreType.DMA,
        ],
    )
    def sc_kernel(tabb, tabs, hv_h, etb, ets,
                  bsrc, bdst, ssrc, sdst, par_h, out_h,
                  acc, sbuf, dbuf, ebuf, gbuf2, dbuf2, sgbuf, dgbuf, exbuf,
                  srow, drow, rowsA, rowsB, obuf, pbuf, sem, sem2, sem3, sem4):
        b = lax.axis_index("c")
        t = lax.axis_index("s")
        bN = b * N
        bN2 = b * N2

        pltpu.sync_copy(par_h, pbuf)

        def zero_rows():
            def zb(i, _):
                for c9 in range(W // L):
                    rowsA[i, pl.ds(L * c9, L)] = jnp.zeros((L,), jnp.float32)
                return 0
            lax.fori_loop(0, NCHK, zb, 0)

        def zero_acc():
            def zc(j, _):
                pltpu.sync_copy(rowsA,
                                acc.at[pl.ds(t * NPT + j * NCHK, NCHK)])
                return 0
            lax.fori_loop(0, NPT // NCHK, zc, 0)

        def edge_phase(tab_ref, et_ref, src2d, dst2d, hs_col,
                       nchunks, sec_off, src_row0, et_row0):
            H = CH // 2

            def issue_src_dst(k):
                pltpu.async_copy(src2d.at[src_row0 + k], sbuf, sem4)
                pltpu.async_copy(dst2d.at[src_row0 + k], dbuf, sem4)

            def issue_et(k):
                pltpu.async_copy(et_ref.at[et_row0 + k], ebuf, sem4)

            def wait_linear():
                pltpu.make_async_copy(src2d.at[src_row0], sbuf, sem4).wait()
                pltpu.make_async_copy(dst2d.at[src_row0], dbuf, sem4).wait()
                pltpu.make_async_copy(et_ref.at[et_row0], ebuf, sem4).wait()

            def scale_half(rowsX, off):
                for j in range(H // L):
                    idr = lax.iota(jnp.int32, L) + L * j
                    exv16 = exbuf[pl.ds(off + L * j, L)]
                    plsc.store_scatter(
                        rowsX, [idr, jnp.full((L,), C, jnp.int32)], exv16)

                @plsc.parallel_loop(0, H, unroll=8)
                def _(e):
                    exv = plsc.load_gather(
                        exbuf, [jnp.full((L,), e + off, jnp.int32)])
                    for c9 in range(C // L):
                        sl2 = pl.ds(L * c9, L)
                        rowsX[e, sl2] = rowsX[e, sl2] * exv

            def wait_scatters():
                pltpu.make_async_copy(rowsA, acc.at[dbuf2.at[0]], sem).wait()
                pltpu.make_async_copy(rowsB, acc.at[dbuf2.at[1]], sem2).wait()

            issue_src_dst(0)
            issue_et(0)

            def chunk(k, _):
                wait_linear()
                for j in range(CH // L):
                    sl = pl.ds(L * j, L)
                    si = sbuf[sl] + sec_off
                    di = dbuf[sl] + sec_off
                    gbuf2[j // 4, pl.ds(L * (j % 4), L)] = si + bN
                    sgbuf[sl] = si + bN
                    dgbuf[sl] = di + bN
                gs = pltpu.async_copy(hv_h.at[sgbuf], srow, sem3)
                gd = pltpu.async_copy(hv_h.at[dgbuf], drow, sem3)
                @pl.when(k > 0)
                def _():
                    wait_scatters()
                for j in range(CH // L):
                    dbuf2[j // 4, pl.ds(L * (j % 4), L)] = dbuf[pl.ds(L * j, L)] + sec_off

                @pl.when(k + 1 < nchunks)
                def _():
                    issue_src_dst(k + 1)
                gA = pltpu.async_copy(tab_ref.at[gbuf2.at[0]], rowsA, sem)
                gB = pltpu.async_copy(tab_ref.at[gbuf2.at[1]], rowsB, sem2)
                gs.wait()
                gd.wait()
                for j in range(CH // L):
                    sl = pl.ds(L * j, L)
                    idr = lax.iota(jnp.int32, L) + L * j
                    hsv = plsc.load_gather(
                        srow, [idr, jnp.full((L,), hs_col, jnp.int32)])
                    hdv = plsc.load_gather(
                        drow, [idr, jnp.full((L,), hs_col + 1, jnp.int32)])
                    al = hsv + hdv + ebuf[sl]
                    al = jnp.maximum(al, 0.2 * al)
                    exbuf[sl] = jnp.exp(al)

                @pl.when(k + 1 < nchunks)
                def _():
                    issue_et(k + 1)
                gA.wait()
                scale_half(rowsA, 0)
                pltpu.async_copy(rowsA, acc.at[dbuf2.at[0]], sem, add=True)
                gB.wait()
                scale_half(rowsB, H)
                pltpu.async_copy(rowsB, acc.at[dbuf2.at[1]], sem2, add=True)
                return 0
            lax.fori_loop(0, nchunks, chunk, 0)
            wait_scatters()

        def finalize(is_bend):
            def fch(jj, _):
                n0 = t * NPT + jj * NCHK
                pltpu.sync_copy(acc.at[pl.ds(n0, NCHK)], rowsA)
                if not is_bend:
                    pltpu.sync_copy(out_h.at[pl.ds(bN2 + n0, NCHK)], obuf)

                @plsc.parallel_loop(0, NCHK, unroll=2)
                def fn(n):
                    denv = plsc.load_gather(
                        rowsA, [jnp.full((L,), n, jnp.int32),
                                jnp.full((L,), C, jnp.int32)])
                    rcp = 1.0 / (denv + 1e-16)
                    for c9 in range(C // L):
                        sl = pl.ds(L * c9, L)
                        bias = pbuf[pl.ds((0 if is_bend else C) + L * c9, L)]
                        v = rowsA[n, sl] * rcp + bias
                        if is_bend:
                            r = jnp.maximum(v, 0.01 * v) * pbuf[pl.ds(2 * C, L)]
                            obuf[n, sl] = r
                        else:
                            obuf[n, sl] = obuf[n, sl] + v * pbuf[pl.ds(2 * C + L, L)]
                pltpu.sync_copy(obuf, out_h.at[pl.ds(bN2 + n0, NCHK)])
                return 0
            lax.fori_loop(0, NPT // NCHK, fch, 0)

        zero_rows()
        zero_acc()
        plsc.subcore_barrier()
        edge_phase(tabb, etb, bsrc, bdst, 0,
                   KB, jnp.int32(0), t * KB, b * (KB * NT) + t * KB)
        plsc.subcore_barrier()
        finalize(True)
        zero_rows()
        zero_acc()
        plsc.subcore_barrier()
        part = t % 2
        sec = t // 2
        edge_phase(tabs, ets, ssrc, sdst, 2,
                   KS, sec * NPS, part * KS, b * (KS * 2) + part * KS)
        plsc.subcore_barrier()
        finalize(False)

    return sc_kernel


def kernel(x, section_edge_index, bend_edge_index, section_edge_attr,
           bend_edge_attr, enc_W, enc_b, gatb_W, gatb_as, gatb_ad, gatb_ae,
           gatb_We, gatb_bias, gats_W, gats_as, gats_ad, gats_ae, gats_We,
           gats_bias, mix_w):
    B, S, NPS, F = x.shape
    N = S * NPS
    C = gatb_W.shape[1]
    ES = section_edge_index.shape[1]
    EB = bend_edge_index.shape[1]
    NT = 16
    KB = -(-(-(-EB // (NT * CH))) // 8) * 8
    EBp = KB * NT * CH
    KS = -(-(-(-ES // (2 * CH))) // 8) * 8
    ESp_half = KS * 2 * CH
    while (B * EBp) % 1024:
        KB += 1
        EBp = KB * NT * CH
    while (B * ESp_half) % 1024:
        KS += 1
        ESp_half = KS * 2 * CH
    ESp = ESp_half

    f32 = jnp.float32
    veb = enc_W @ (gatb_We @ gatb_ae)
    ceb = enc_b @ (gatb_We @ gatb_ae)
    ves = enc_W @ (gats_We @ gats_ae)
    ces = enc_b @ (gats_We @ gats_ae)
    v5b = jnp.concatenate([veb, ceb[None]])[None, :]
    v5s = jnp.concatenate([ves, ces[None]])[None, :]
    ab = jnp.concatenate([gatb_as[:, None], gatb_ad[:, None]], axis=1)
    as_ = jnp.concatenate([gats_as[:, None], gats_ad[:, None]], axis=1)
    w = jax.nn.softmax(mix_w)
    params = jnp.concatenate([gatb_bias, gats_bias,
                              jnp.full((L,), w[0], f32),
                              jnp.full((L,), w[1], f32)])

    x2d = x.reshape(B * N, F)
    tabb, tabs, hv = _build_tables(x2d, gatb_W, gats_W, ab, as_)

    def pad_ea(ea, Ep):
        E = ea.shape[1]
        eap = jnp.pad(ea, ((0, 0), (0, Ep - E), (0, 0)))
        flat = eap.reshape(B * Ep, 4)
        return jnp.concatenate([flat, jnp.ones((B * Ep, 1), f32)], axis=1)
    etb = _edge_terms(pad_ea(bend_edge_attr, EBp), v5b, EBp, EB)
    ets = _edge_terms(pad_ea(section_edge_attr, ESp), v5s, ESp, ES)

    def pad_idx(idx, Ep):
        E = idx.shape[0]
        return jnp.pad(idx, (0, Ep - E)).astype(jnp.int32).reshape(Ep // CH, CH)
    bsrc = pad_idx(bend_edge_index[0], EBp)
    bdst = pad_idx(bend_edge_index[1], EBp)
    ssrc = pad_idx(section_edge_index[0], ESp)
    sdst = pad_idx(section_edge_index[1], ESp)

    sc = _make_sc_kernel(B, N, NPS, C, KB, KS)
    out = sc(tabb, tabs, hv, etb, ets,
             bsrc, bdst, ssrc, sdst, params)
    N2 = out.shape[0] // B
    return out.reshape(B, N2, C)[:, :N]
